# Initial kernel scaffold; baseline (speedup 1.0000x reference)
import jax
import jax.numpy as jnp
from jax.experimental import pallas as pl


def kernel(token_ids, table):
    raise NotImplementedError("write your pallas kernel here")


# speedup vs baseline: 5.7502x; 5.7502x over previous
import functools

import jax
import jax.numpy as jnp
from jax import lax
from jax.experimental import pallas as pl
from jax.experimental.pallas import tpu as pltpu
from jax.experimental.pallas import tpu_sc as plsc

D = 128
NC = 2
NS = 16
NW = NC * NS
CH = 128


@functools.partial(jax.jit, static_argnums=(2,))
def _gather(idx3d, table, n_chunks):
    mesh = plsc.VectorSubcoreMesh(core_axis_name="c", subcore_axis_name="s")
    bpw = n_chunks * CH

    @functools.partial(
        pl.kernel,
        out_type=jax.ShapeDtypeStruct((NW, bpw, D), jnp.float32),
        mesh=mesh,
        scratch_types=[
            pltpu.VMEM((n_chunks, CH), jnp.int32),
            pltpu.VMEM((CH, D), jnp.float32),
            pltpu.SemaphoreType.DMA,
        ],
    )
    def k(idx_hbm, table_hbm, out_hbm, idx_v, rows_v, sem):
        wid = lax.axis_index("s") * NC + lax.axis_index("c")
        pltpu.sync_copy(idx_hbm.at[wid], idx_v)

        @pl.loop(0, n_chunks)
        def _body(c):
            pltpu.async_copy(table_hbm.at[idx_v.at[c]], rows_v, sem).wait()
            pltpu.sync_copy(rows_v, out_hbm.at[wid, pl.ds(c * CH, CH)])

    return k(idx3d, table)


def kernel(token_ids, table):
    s, t = token_ids.shape
    b = s * t
    n_chunks = b // (NW * CH)
    idx3d = token_ids.reshape(NW, n_chunks, CH).astype(jnp.int32)
    out = _gather(idx3d, table, n_chunks)
    return out.reshape(s, t, D)


# speedup vs baseline: 7.2786x; 1.2658x over previous
import functools

import jax
import jax.numpy as jnp
from jax import lax
from jax.experimental import pallas as pl
from jax.experimental.pallas import tpu as pltpu
from jax.experimental.pallas import tpu_sc as plsc

D = 128
NC = 2
NS = 16
NW = NC * NS
CH = 128


@functools.partial(jax.jit, static_argnums=(2,))
def _gather(idx3d, table, n_chunks):
    assert n_chunks % 2 == 0
    mesh = plsc.VectorSubcoreMesh(core_axis_name="c", subcore_axis_name="s")
    bpw = n_chunks * CH

    @functools.partial(
        pl.kernel,
        out_type=jax.ShapeDtypeStruct((NW, bpw, D), jnp.float32),
        mesh=mesh,
        scratch_types=[
            pltpu.VMEM((n_chunks, CH), jnp.int32),
            pltpu.VMEM((CH, D), jnp.float32),
            pltpu.VMEM((CH, D), jnp.float32),
            pltpu.SemaphoreType.DMA,
            pltpu.SemaphoreType.DMA,
            pltpu.SemaphoreType.DMA,
            pltpu.SemaphoreType.DMA,
        ],
    )
    def k(idx_hbm, table_hbm, out_hbm, idx_v, rows_a, rows_b, gsa, gsb, wsa, wsb):
        wid = lax.axis_index("s") * NC + lax.axis_index("c")
        pltpu.sync_copy(idx_hbm.at[wid], idx_v)

        def gather(c, buf, sem):
            return pltpu.make_async_copy(table_hbm.at[idx_v.at[c]], buf, sem)

        def wb(c, buf, sem):
            return pltpu.make_async_copy(
                buf, out_hbm.at[wid, pl.ds(c * CH, CH)], sem)

        gather(0, rows_a, gsa).start()
        gather(1, rows_b, gsb).start()

        @pl.loop(0, n_chunks, step=2)
        def _body(c):
            gather(0, rows_a, gsa).wait()
            wb(c, rows_a, wsa).start()
            gather(0, rows_b, gsb).wait()
            wb(c + 1, rows_b, wsb).start()
            wb(0, rows_a, wsa).wait()

            @pl.when(c + 2 < n_chunks)
            def _refill_a():
                gather(c + 2, rows_a, gsa).start()

            wb(0, rows_b, wsb).wait()

            @pl.when(c + 3 < n_chunks)
            def _refill_b():
                gather(c + 3, rows_b, gsb).start()

    return k(idx3d, table)


def kernel(token_ids, table):
    s, t = token_ids.shape
    b = s * t
    n_chunks = b // (NW * CH)
    idx3d = token_ids.reshape(NW, n_chunks, CH).astype(jnp.int32)
    out = _gather(idx3d, table, n_chunks)
    return out.reshape(s, t, D)


# speedup vs baseline: 7.7998x; 1.0716x over previous
import functools

import jax
import jax.numpy as jnp
from jax import lax
from jax.experimental import pallas as pl
from jax.experimental.pallas import tpu as pltpu
from jax.experimental.pallas import tpu_sc as plsc

D = 128
NC = 2
NS = 16
NW = NC * NS
CH = 128
NBUF = 5


@functools.partial(jax.jit, static_argnums=(2,))
def _gather(idx3d, table, n_chunks):
    assert n_chunks % NBUF == 0
    mesh = plsc.VectorSubcoreMesh(core_axis_name="c", subcore_axis_name="s")
    bpw = n_chunks * CH

    @functools.partial(
        pl.kernel,
        out_type=jax.ShapeDtypeStruct((NW, bpw, D), jnp.float32),
        mesh=mesh,
        scratch_types=(
            [pltpu.VMEM((n_chunks, CH), jnp.int32)]
            + [pltpu.VMEM((CH, D), jnp.float32)] * NBUF
            + [pltpu.SemaphoreType.DMA] * (2 * NBUF)
        ),
    )
    def k(idx_hbm, table_hbm, out_hbm, idx_v, *bufs_and_sems):
        bufs = bufs_and_sems[:NBUF]
        gs = bufs_and_sems[NBUF:2 * NBUF]
        ws = bufs_and_sems[2 * NBUF:]
        wid = lax.axis_index("s") * NC + lax.axis_index("c")
        pltpu.sync_copy(idx_hbm.at[wid], idx_v)

        def gather(c, buf, sem):
            return pltpu.make_async_copy(table_hbm.at[idx_v.at[c]], buf, sem)

        def wb(c, buf, sem):
            return pltpu.make_async_copy(
                buf, out_hbm.at[wid, pl.ds(c * CH, CH)], sem)

        for j in range(NBUF):
            gather(j, bufs[j], gs[j]).start()

        @pl.loop(0, n_chunks, step=NBUF)
        def _body(c0):
            for j in range(NBUF):
                gather(0, bufs[j], gs[j]).wait()
                wb(c0 + j, bufs[j], ws[j]).start()
            for j in range(NBUF):
                wb(0, bufs[j], ws[j]).wait()

                @pl.when(c0 + NBUF + j < n_chunks)
                def _refill(j=j):
                    gather(c0 + NBUF + j, bufs[j], gs[j]).start()

    return k(idx3d, table)


def kernel(token_ids, table):
    s, t = token_ids.shape
    b = s * t
    n_chunks = b // (NW * CH)
    idx3d = token_ids.reshape(NW, n_chunks, CH).astype(jnp.int32)
    out = _gather(idx3d, table, n_chunks)
    return out.reshape(s, t, D)


# speedup vs baseline: 7.8681x; 1.0088x over previous
import functools

import jax
import jax.numpy as jnp
from jax import lax
from jax.experimental import pallas as pl
from jax.experimental.pallas import tpu as pltpu
from jax.experimental.pallas import tpu_sc as plsc

D = 128
NC = 2
NS = 16
NW = NC * NS
CH = 64
NBUF = 10


@functools.partial(jax.jit, static_argnums=(2,))
def _gather(idx3d, table, n_chunks):
    assert n_chunks % NBUF == 0
    mesh = plsc.VectorSubcoreMesh(core_axis_name="c", subcore_axis_name="s")
    bpw = n_chunks * CH

    @functools.partial(
        pl.kernel,
        out_type=jax.ShapeDtypeStruct((NW, bpw, D), jnp.float32),
        mesh=mesh,
        scratch_types=(
            [pltpu.VMEM((n_chunks, CH), jnp.int32)]
            + [pltpu.VMEM((CH, D), jnp.float32)] * NBUF
            + [pltpu.SemaphoreType.DMA] * (2 * NBUF)
        ),
    )
    def k(idx_hbm, table_hbm, out_hbm, idx_v, *bufs_and_sems):
        bufs = bufs_and_sems[:NBUF]
        gs = bufs_and_sems[NBUF:2 * NBUF]
        ws = bufs_and_sems[2 * NBUF:]
        wid = lax.axis_index("s") * NC + lax.axis_index("c")
        pltpu.sync_copy(idx_hbm.at[wid], idx_v)

        def gather(c, buf, sem):
            return pltpu.make_async_copy(table_hbm.at[idx_v.at[c]], buf, sem)

        def wb(c, buf, sem):
            return pltpu.make_async_copy(
                buf, out_hbm.at[wid, pl.ds(c * CH, CH)], sem)

        for j in range(NBUF):
            gather(j, bufs[j], gs[j]).start()

        @pl.loop(0, n_chunks, step=NBUF)
        def _body(c0):
            for j in range(NBUF):
                gather(0, bufs[j], gs[j]).wait()
                wb(c0 + j, bufs[j], ws[j]).start()
            for j in range(NBUF):
                wb(0, bufs[j], ws[j]).wait()

                @pl.when(c0 + NBUF + j < n_chunks)
                def _refill(j=j):
                    gather(c0 + NBUF + j, bufs[j], gs[j]).start()

    return k(idx3d, table)


def kernel(token_ids, table):
    s, t = token_ids.shape
    b = s * t
    n_chunks = b // (NW * CH)
    idx3d = token_ids.reshape(NW, n_chunks, CH).astype(jnp.int32)
    out = _gather(idx3d, table, n_chunks)
    return out.reshape(s, t, D)
